# baseline (speedup 1.0000x reference)
import functools

import jax
import jax.numpy as jnp
from jax import lax
from jax.experimental import pallas as pl
from jax.experimental.pallas import tpu as pltpu
from jax.experimental.pallas import tpu_sc as plsc

N = 10000
E = 320000
D = 128
NC = 2
NS = 16
NW = NC * NS
CH = 80
NCHUNK = 128
EPT = CH * NCHUNK
EPAD = NW * EPT
NP = 10112
RPT = NP // NS


def _seg_sum_body(feat_hbm, srcp_hbm, dstp_hbm, zeros_hbm, out_hbm, agg_sh,
                  is0, is1, is2, is3, id0, id1, id2, id3, r0, r1, r2, r3,
                  si0, si1, si2, si3, sg0, sg1, sg2, sg3):
    c = lax.axis_index("c")
    s = lax.axis_index("s")
    ebase = (c * NS + s) * EPT

    IS = (is0, is1, is2, is3)
    ID = (id0, id1, id2, id3)
    RW = (r0, r1, r2, r3)
    SI = (si0, si1, si2, si3)
    SG = (sg0, sg1, sg2, sg3)

    def idx_cps(j, k):
        return (pltpu.make_async_copy(
                    srcp_hbm.at[pl.ds(ebase + j * CH, CH)], IS[k], SI[k]),
                pltpu.make_async_copy(
                    dstp_hbm.at[pl.ds(ebase + j * CH, CH)], ID[k], SI[k]))

    def fire_idx(j, k):
        for cp in idx_cps(j, k):
            cp.start()

    def wait_idx(j, k):
        for cp in idx_cps(j, k):
            cp.wait()

    def gat_cp(k):
        return pltpu.make_async_copy(feat_hbm.at[IS[k]], RW[k], SG[k])

    fire_idx(0, 0)
    fire_idx(1, 1)
    pltpu.sync_copy(zeros_hbm.at[pl.ds(s * RPT, RPT)],
                    agg_sh.at[pl.ds(s * RPT, RPT)])
    wait_idx(0, 0)
    gat_cp(0).start()
    fire_idx(2, 2)
    wait_idx(1, 1)
    gat_cp(1).start()
    fire_idx(3, 3)
    wait_idx(2, 2)
    gat_cp(2).start()
    plsc.subcore_barrier()

    @pl.loop(0, NCHUNK, step=4)
    def chunk(i):
        for k in range(4):
            j = i + k
            gat_cp(k).wait()
            pltpu.sync_copy(RW[k], agg_sh.at[ID[k]], add=True)

            @pl.when(j + 4 < NCHUNK)
            def _():
                fire_idx(j + 4, k)

            @pl.when(j + 3 < NCHUNK)
            def _():
                wait_idx(j + 3, (k + 3) % 4)
                gat_cp((k + 3) % 4).start()

    plsc.subcore_barrier()

    pltpu.sync_copy(agg_sh.at[pl.ds(s * RPT, RPT)],
                    out_hbm.at[pl.ds(c * NP + s * RPT, RPT)])


_seg_sum = pl.kernel(
    _seg_sum_body,
    out_type=jax.ShapeDtypeStruct((NC * NP, D), jnp.float32),
    mesh=plsc.VectorSubcoreMesh(core_axis_name="c", subcore_axis_name="s",
                                num_cores=NC, num_subcores=NS),
    scratch_types=(
        [pltpu.VMEM_SHARED((NP, D), jnp.float32)]
        + [pltpu.VMEM((CH,), jnp.int32) for _ in range(8)]
        + [pltpu.VMEM((CH, D), jnp.float32) for _ in range(4)]
        + [pltpu.SemaphoreType.DMA for _ in range(8)]
    ),
)

BM = 2000


def _mlp_body(relu, p0_ref, p1_ref, x_ref, w_ref, o_ref):
    acc = p0_ref[...] + p1_ref[...] + x_ref[...]
    y = lax.dot_general(acc, w_ref[...], (((1,), (1,)), ((), ())),
                        preferred_element_type=jnp.float32)
    o_ref[...] = jnp.maximum(y, 0.0) if relu else y


def _mlp(p0, p1, x, w, relu):
    dout = w.shape[0]
    return pl.pallas_call(
        functools.partial(_mlp_body, relu),
        grid=(N // BM,),
        in_specs=[
            pl.BlockSpec((BM, D), lambda i: (i, 0)),
            pl.BlockSpec((BM, D), lambda i: (i, 0)),
            pl.BlockSpec((BM, D), lambda i: (i, 0)),
            pl.BlockSpec((dout, D), lambda i: (0, 0)),
        ],
        out_specs=pl.BlockSpec((BM, dout), lambda i: (i, 0)),
        out_shape=jax.ShapeDtypeStruct((N, dout), jnp.float32),
    )(p0, p1, x, w)


@jax.jit
def kernel(x, edge_index, W1, W2):
    src = edge_index[0]
    dst = edge_index[1]
    pad = EPAD - E
    srcp = jnp.concatenate([src, jnp.zeros((pad,), jnp.int32)])
    dstp = jnp.concatenate([dst, jnp.full((pad,), NP - 1, jnp.int32)])
    zeros = jnp.zeros((NP, D), jnp.float32)
    p1 = _seg_sum(x, srcp, dstp, zeros)
    h = _mlp(p1[:N], p1[NP:NP + N], x, W1, relu=True)
    p2 = _seg_sum(h, srcp, dstp, zeros)
    out = _mlp(p2[:N], p2[NP:NP + N], h, W2, relu=False)
    return out


# speedup vs baseline: 2.8794x; 2.8794x over previous
import functools

import jax
import jax.numpy as jnp
from jax import lax
from jax.experimental import pallas as pl
from jax.experimental.pallas import tpu as pltpu
from jax.experimental.pallas import tpu_sc as plsc

N = 10000
E = 320000
D = 128
NC = 2
NS = 16
NW = NC * NS
CH = 88
NCHUNK = 114
EPT = CH * NCHUNK
EPAD = NW * EPT
NP = 10112
RPT = NP // NS


def _seg_sum_body(feat_hbm, srcp_hbm, dstp_hbm, zeros_hbm, out_hbm, agg_sh,
                  is0, is1, is2, id0, id1, id2, r0, r1, r2,
                  si0, si1, si2, sg0, sg1, sg2):
    c = lax.axis_index("c")
    s = lax.axis_index("s")
    ebase = (c * NS + s) * EPT

    IS = (is0, is1, is2)
    ID = (id0, id1, id2)
    RW = (r0, r1, r2)
    SI = (si0, si1, si2)
    SG = (sg0, sg1, sg2)

    def idx_cps(j, k):
        return (pltpu.make_async_copy(
                    srcp_hbm.at[pl.ds(ebase + j * CH, CH)], IS[k], SI[k]),
                pltpu.make_async_copy(
                    dstp_hbm.at[pl.ds(ebase + j * CH, CH)], ID[k], SI[k]))

    def fire_idx(j, k):
        for cp in idx_cps(j, k):
            cp.start()

    def wait_idx(j, k):
        for cp in idx_cps(j, k):
            cp.wait()

    def gat_cp(k):
        return pltpu.make_async_copy(feat_hbm.at[IS[k]], RW[k], SG[k])

    fire_idx(0, 0)
    fire_idx(1, 1)
    pltpu.sync_copy(zeros_hbm.at[pl.ds(s * RPT, RPT)],
                    agg_sh.at[pl.ds(s * RPT, RPT)])
    wait_idx(0, 0)
    gat_cp(0).start()
    fire_idx(2, 2)
    wait_idx(1, 1)
    gat_cp(1).start()
    plsc.subcore_barrier()

    @pl.loop(0, NCHUNK, step=3)
    def chunk(i):
        for k in range(3):
            j = i + k
            gat_cp(k).wait()
            pltpu.sync_copy(RW[k], agg_sh.at[ID[k]], add=True)

            @pl.when(j + 3 < NCHUNK)
            def _():
                fire_idx(j + 3, k)

            @pl.when(j + 2 < NCHUNK)
            def _():
                wait_idx(j + 2, (k + 2) % 3)
                gat_cp((k + 2) % 3).start()

    plsc.subcore_barrier()

    pltpu.sync_copy(agg_sh.at[pl.ds(s * RPT, RPT)],
                    out_hbm.at[pl.ds(c * NP + s * RPT, RPT)])


_seg_sum = pl.kernel(
    _seg_sum_body,
    out_type=jax.ShapeDtypeStruct((NC * NP, D), jnp.float32),
    mesh=plsc.VectorSubcoreMesh(core_axis_name="c", subcore_axis_name="s",
                                num_cores=NC, num_subcores=NS),
    scratch_types=(
        [pltpu.VMEM_SHARED((NP, D), jnp.float32)]
        + [pltpu.VMEM((CH,), jnp.int32) for _ in range(6)]
        + [pltpu.VMEM((CH, D), jnp.float32) for _ in range(3)]
        + [pltpu.SemaphoreType.DMA for _ in range(6)]
    ),
)

BM = 2000


def _mlp_body(relu, p0_ref, p1_ref, x_ref, w_ref, o_ref):
    acc = p0_ref[...] + p1_ref[...] + x_ref[...]
    y = lax.dot_general(acc, w_ref[...], (((1,), (1,)), ((), ())),
                        preferred_element_type=jnp.float32)
    o_ref[...] = jnp.maximum(y, 0.0) if relu else y


def _mlp(p0, p1, x, w, relu):
    dout = w.shape[0]
    return pl.pallas_call(
        functools.partial(_mlp_body, relu),
        grid=(N // BM,),
        in_specs=[
            pl.BlockSpec((BM, D), lambda i: (i, 0)),
            pl.BlockSpec((BM, D), lambda i: (i, 0)),
            pl.BlockSpec((BM, D), lambda i: (i, 0)),
            pl.BlockSpec((dout, D), lambda i: (0, 0)),
        ],
        out_specs=pl.BlockSpec((BM, dout), lambda i: (i, 0)),
        out_shape=jax.ShapeDtypeStruct((N, dout), jnp.float32),
    )(p0, p1, x, w)


@jax.jit
def kernel(x, edge_index, W1, W2):
    src = edge_index[0]
    dst = edge_index[1]
    pad = EPAD - E
    srcp = jnp.concatenate([src, jnp.zeros((pad,), jnp.int32)])
    dstp = jnp.concatenate([dst, jnp.full((pad,), NP - 1, jnp.int32)])
    zeros = jnp.zeros((NP, D), jnp.float32)
    p1 = _seg_sum(x, srcp, dstp, zeros)
    h = _mlp(p1[:N], p1[NP:NP + N], x, W1, relu=True)
    p2 = _seg_sum(h, srcp, dstp, zeros)
    out = _mlp(p2[:N], p2[NP:NP + N], h, W2, relu=False)
    return out
